# baseline (speedup 1.0000x reference)
import jax
import jax.numpy as jnp
from jax.experimental import pallas as pl
from jax.experimental.pallas import tpu as pltpu

TN = 256
NBUF = 4


def _matmul_kernel(x_ref, w_hbm, o_hbm, w_buf, o_buf, w_sems, o_sems):
    n_out = w_hbm.shape[0]
    nt = n_out // TN

    def w_copy(j, slot):
        return pltpu.make_async_copy(
            w_hbm.at[pl.ds(j * TN, TN), :], w_buf.at[slot], w_sems.at[slot]
        )

    def o_copy(j, slot):
        return pltpu.make_async_copy(
            o_buf.at[slot], o_hbm.at[:, pl.ds(j * TN, TN)], o_sems.at[slot]
        )

    for s in range(min(NBUF, nt)):
        w_copy(s, s).start()

    for j in range(nt):
        slot = j % NBUF
        w_copy(j, slot).wait()
        ob = j % 2
        if j >= 2:
            o_copy(j - 2, ob).wait()
        o_buf[ob] = jax.lax.dot_general(
            x_ref[...], w_buf[slot],
            dimension_numbers=(((1,), (1,)), ((), ())),
            preferred_element_type=jnp.float32,
        )
        o_copy(j, ob).start()
        if j + NBUF < nt:
            w_copy(j + NBUF, slot).start()

    o_copy(nt - 2, nt % 2).wait()
    o_copy(nt - 1, (nt + 1) % 2).wait()


@jax.jit
def kernel(X, W):
    batch, n_in = X.shape
    n_out = W.shape[0]
    return pl.pallas_call(
        _matmul_kernel,
        in_specs=[
            pl.BlockSpec((batch, n_in), lambda: (0, 0)),
            pl.BlockSpec(memory_space=pltpu.MemorySpace.HBM),
        ],
        out_specs=pl.BlockSpec(memory_space=pltpu.MemorySpace.HBM),
        out_shape=jax.ShapeDtypeStruct((batch, n_out), jnp.float32),
        scratch_shapes=[
            pltpu.VMEM((NBUF, TN, n_in), jnp.float32),
            pltpu.VMEM((2, batch, TN), jnp.float32),
            pltpu.SemaphoreType.DMA((NBUF,)),
            pltpu.SemaphoreType.DMA((2,)),
        ],
    )(X, W)


# speedup vs baseline: 1.0455x; 1.0455x over previous
import jax
import jax.numpy as jnp
from jax.experimental import pallas as pl
from jax.experimental.pallas import tpu as pltpu

TN = 512
NBUF = 3


def _matmul_kernel(x_ref, w_hbm, o_hbm, w_buf, o_buf, w_sems, o_sems):
    n_out = w_hbm.shape[0]
    nt = n_out // TN

    def w_copy(j, slot):
        return pltpu.make_async_copy(
            w_hbm.at[pl.ds(j * TN, TN), :], w_buf.at[slot], w_sems.at[slot]
        )

    def o_copy(j, slot):
        return pltpu.make_async_copy(
            o_buf.at[slot], o_hbm.at[:, pl.ds(j * TN, TN)], o_sems.at[slot]
        )

    for s in range(min(NBUF, nt)):
        w_copy(s, s).start()

    for j in range(nt):
        slot = j % NBUF
        w_copy(j, slot).wait()
        ob = j % 2
        if j >= 2:
            o_copy(j - 2, ob).wait()
        o_buf[ob] = jax.lax.dot_general(
            x_ref[...], w_buf[slot],
            dimension_numbers=(((1,), (1,)), ((), ())),
            preferred_element_type=jnp.float32,
        )
        o_copy(j, ob).start()
        if j + NBUF < nt:
            w_copy(j + NBUF, slot).start()

    o_copy(nt - 2, nt % 2).wait()
    o_copy(nt - 1, (nt + 1) % 2).wait()


@jax.jit
def kernel(X, W):
    batch, n_in = X.shape
    n_out = W.shape[0]
    return pl.pallas_call(
        _matmul_kernel,
        in_specs=[
            pl.BlockSpec((batch, n_in), lambda: (0, 0)),
            pl.BlockSpec(memory_space=pltpu.MemorySpace.HBM),
        ],
        out_specs=pl.BlockSpec(memory_space=pltpu.MemorySpace.HBM),
        out_shape=jax.ShapeDtypeStruct((batch, n_out), jnp.float32),
        scratch_shapes=[
            pltpu.VMEM((NBUF, TN, n_in), jnp.float32),
            pltpu.VMEM((2, batch, TN), jnp.float32),
            pltpu.SemaphoreType.DMA((NBUF,)),
            pltpu.SemaphoreType.DMA((2,)),
        ],
    )(X, W)
